# baseline (speedup 1.0000x reference)
import functools

import jax
import jax.numpy as jnp
import numpy as np
from jax.experimental import pallas as pl

_NUM_BINS = 12
_ANGLE_PER_BIN = 2.0 * np.pi / _NUM_BINS
_TWO_PI = 2.0 * np.pi


def _rpn_kernel(x_ref, w1_ref, b1_ref, w2_ref, b2_ref,
                wobj_ref, bobj_ref, wbox_ref, bbox_ref,
                whcls_ref, bhcls_ref, whd_ref, bhd_ref, out_ref):
    x = x_ref[...]
    h = jnp.maximum(
        jax.lax.dot_general(w1_ref[...], x, (((0,), (1,)), ((), ())),
                            preferred_element_type=jnp.float32)
        + b1_ref[...], 0.0)
    h = jnp.maximum(
        jax.lax.dot_general(w2_ref[...], h, (((0,), (0,)), ((), ())),
                            preferred_element_type=jnp.float32)
        + b2_ref[...], 0.0)

    def head(w_ref, b_ref):
        return (jax.lax.dot_general(w_ref[...], h, (((0,), (0,)), ((), ())),
                                    preferred_element_type=jnp.float32)
                + b_ref[...])

    obj = jax.nn.sigmoid(head(wobj_ref, bobj_ref))
    d = head(wbox_ref, bbox_ref)
    hcls = head(whcls_ref, bhcls_ref)
    hd = head(whd_ref, bhd_ref)

    mx = jnp.max(hcls, axis=0, keepdims=True)
    iota = jax.lax.broadcasted_iota(jnp.int32, hcls.shape, 0)
    idx = jnp.min(jnp.where(hcls == mx, iota, _NUM_BINS),
                  axis=0, keepdims=True)
    delta = jnp.sum(jnp.where(iota == idx, hd, 0.0), axis=0, keepdims=True)
    ang = jnp.mod(idx.astype(jnp.float32) * _ANGLE_PER_BIN + delta, _TWO_PI)

    out_ref[...] = jnp.concatenate([obj, ang, d], axis=0)


@functools.partial(jax.jit, static_argnames=())
def kernel(voted_xyz, voted_features, W1, b1, W2, b2, W_obj, b_obj,
           W_box, b_box, W_hcls, b_hcls, W_hd, b_hd):
    B, N, C = voted_features.shape
    H = W1.shape[1]
    M = B * N
    T = 4096
    grid = (M // T,)

    x = voted_features.reshape(M, C)

    def const(shape):
        ndim = len(shape)
        return pl.BlockSpec(shape, lambda i: (0,) * ndim)

    out = pl.pallas_call(
        _rpn_kernel,
        grid=grid,
        in_specs=[
            pl.BlockSpec((T, C), lambda i: (i, 0)),
            const((C, H)), const((H, 1)),
            const((H, H)), const((H, 1)),
            const((H, 1)), const((1, 1)),
            const((H, 6)), const((6, 1)),
            const((H, _NUM_BINS)), const((_NUM_BINS, 1)),
            const((H, _NUM_BINS)), const((_NUM_BINS, 1)),
        ],
        out_specs=pl.BlockSpec((8, T), lambda i: (0, i)),
        out_shape=jax.ShapeDtypeStruct((8, M), jnp.float32),
    )(x, W1, b1.reshape(H, 1), W2, b2.reshape(H, 1),
      W_obj, b_obj.reshape(1, 1), W_box, b_box.reshape(6, 1),
      W_hcls, b_hcls.reshape(_NUM_BINS, 1), W_hd, b_hd.reshape(_NUM_BINS, 1))

    obj = out[0].reshape(B, N)
    ang = out[1].reshape(B, N)
    d = out[2:8].T
    xyz = voted_xyz.reshape(M, 3)
    boxes = jnp.concatenate([xyz - d[:, 0:3], xyz + d[:, 3:6]],
                            axis=-1).reshape(B, N, 6)
    return (obj, boxes, ang)


# speedup vs baseline: 1.2328x; 1.2328x over previous
import functools

import jax
import jax.numpy as jnp
import numpy as np
from jax.experimental import pallas as pl

_NUM_BINS = 12
_ANGLE_PER_BIN = 2.0 * np.pi / _NUM_BINS
_TWO_PI = 2.0 * np.pi


def _rpn_kernel(x_ref, w_ref, out_ref):
    x = x_ref[...]
    w1 = w_ref[0:256, :]
    w2 = w_ref[256:384, :]
    wh = w_ref[384:512, 0:32]
    b1 = w_ref[384:512, 32:33]
    b2 = w_ref[384:512, 33:34]
    bh = w_ref[384:416, 34:35]

    h = jnp.maximum(
        jax.lax.dot_general(w1, x, (((0,), (1,)), ((), ())),
                            preferred_element_type=jnp.float32)
        + b1, 0.0)
    h = jnp.maximum(
        jax.lax.dot_general(w2, h, (((0,), (0,)), ((), ())),
                            preferred_element_type=jnp.float32)
        + b2, 0.0)
    o = (jax.lax.dot_general(wh, h, (((0,), (0,)), ((), ())),
                             preferred_element_type=jnp.float32)
         + bh)

    obj = jax.nn.sigmoid(o[0:1, :])

    hcls = o[7:7 + _NUM_BINS, :]
    hd = o[7 + _NUM_BINS:7 + 2 * _NUM_BINS, :]
    mx = jnp.max(hcls, axis=0, keepdims=True)
    iota = jax.lax.broadcasted_iota(jnp.int32, hcls.shape, 0)
    idx = jnp.min(jnp.where(hcls == mx, iota, _NUM_BINS),
                  axis=0, keepdims=True)
    delta = jnp.sum(jnp.where(iota == idx, hd, 0.0), axis=0, keepdims=True)
    ang = jnp.mod(idx.astype(jnp.float32) * _ANGLE_PER_BIN + delta, _TWO_PI)

    out_ref[...] = jnp.concatenate([obj, ang, o[1:7, :]], axis=0)


@functools.partial(jax.jit, static_argnames=())
def kernel(voted_xyz, voted_features, W1, b1, W2, b2, W_obj, b_obj,
           W_box, b_box, W_hcls, b_hcls, W_hd, b_hd):
    B, N, C = voted_features.shape
    H = W1.shape[1]
    M = B * N
    T = 4096
    grid = (M // T,)

    x = voted_features.reshape(M, C)

    wh = jnp.concatenate([W_obj, W_box, W_hcls, W_hd,
                          jnp.zeros((H, 1), dtype=W_obj.dtype)], axis=1)
    bh = jnp.concatenate([b_obj, b_box, b_hcls, b_hd,
                          jnp.zeros((H - 31,), dtype=b_obj.dtype)], axis=0)
    blk = jnp.concatenate(
        [wh, b1[:, None], b2[:, None], bh[:, None],
         jnp.zeros((H, H - 35), dtype=W_obj.dtype)], axis=1)
    packed = jnp.concatenate([W1, W2, blk], axis=0)

    out = pl.pallas_call(
        _rpn_kernel,
        grid=grid,
        in_specs=[
            pl.BlockSpec((T, C), lambda i: (i, 0)),
            pl.BlockSpec((512, H), lambda i: (0, 0)),
        ],
        out_specs=pl.BlockSpec((8, T), lambda i: (0, i)),
        out_shape=jax.ShapeDtypeStruct((8, M), jnp.float32),
    )(x, packed)

    obj = out[0].reshape(B, N)
    ang = out[1].reshape(B, N)
    d = out[2:8].T
    xyz = voted_xyz.reshape(M, 3)
    boxes = jnp.concatenate([xyz - d[:, 0:3], xyz + d[:, 3:6]],
                            axis=-1).reshape(B, N, 6)
    return (obj, boxes, ang)
